# baseline (speedup 1.0000x reference)
import functools

import jax
import jax.numpy as jnp
from jax import lax
from jax.experimental import pallas as pl
from jax.experimental.pallas import tpu as pltpu
from jax.experimental.pallas import tpu_sc as plsc

_N = 3200000
_G = 8
_NW = 32
_PER_W = _N // _NW
_CH = 20000
_CHS = (2000, 6000, 12000, 20000, 20000, 20000, 20000)
_OFFS = tuple(sum(_CHS[:i]) for i in range(len(_CHS)))
_NCH = len(_CHS)
_L = 16
_U = 25
_STR = 1568
_NPR = 64
_WIN = 1600
_SB = _L * _U
_NSNAP = _PER_W // _SB

_mesh = plsc.VectorSubcoreMesh(core_axis_name="c", subcore_axis_name="s")


@functools.partial(
    pl.kernel,
    mesh=_mesh,
    compiler_params=pltpu.CompilerParams(needs_layout_passes=False),
    out_type=[
        jax.ShapeDtypeStruct((_NW, 128), jnp.float32),
        jax.ShapeDtypeStruct((_NW, _L), jnp.int32),
    ],
    scratch_types=[
        pltpu.VMEM((_CH,), jnp.float32),
        pltpu.VMEM((_CH,), jnp.float32),
        pltpu.VMEM((_CH,), jnp.float32),
        pltpu.VMEM((_CH,), jnp.float32),
        pltpu.VMEM((_NPR * _L,), jnp.int32),
        pltpu.VMEM((7 * _WIN,), jnp.int32),
        pltpu.VMEM(((_NSNAP + 6) * _L,), jnp.float32),
        pltpu.VMEM((7 * _SB,), jnp.float32),
        pltpu.VMEM((7 * _SB,), jnp.float32),
        pltpu.VMEM((_L,), jnp.int32),
        pltpu.VMEM((128,), jnp.float32),
        pltpu.SemaphoreType.DMA,
        pltpu.SemaphoreType.DMA,
        pltpu.SemaphoreType.DMA,
    ],
)
def _sc_partials(p_hbm, t_hbm, lab_hbm, sums_out, bnds_out,
                 pbuf0, tbuf0, pbuf1, tbuf1,
                 sampbuf, winbuf, snap, rpbuf, rtbuf, bscr, sacc,
                 sem0, sem1, semx):
    wid = lax.axis_index("s") * 2 + lax.axis_index("c")
    base = wid * _PER_W
    zeros = jnp.zeros((_L,), jnp.float32)
    lane = lax.iota(jnp.int32, _L)
    snap[pl.ds(0, _L)] = zeros

    probe_hs = [pltpu.async_copy(lab_hbm.at[pl.ds(base + k * _STR, _L)],
                                 sampbuf.at[pl.ds(k * _L, _L)], semx)
                for k in range(_NPR)]

    slots = ((pbuf0, tbuf0, sem0), (pbuf1, tbuf1, sem1))
    _NS = len(slots)

    def start_chunk(ci):
        pb, tb, sem = slots[ci % _NS]
        off = base + _OFFS[ci]
        sz = _CHS[ci]
        return (pltpu.async_copy(p_hbm.at[pl.ds(off, sz)],
                                 pb.at[pl.ds(0, sz)], sem),
                pltpu.async_copy(t_hbm.at[pl.ds(off, sz)],
                                 tb.at[pl.ds(0, sz)], sem))

    handles = {ci: start_chunk(ci) for ci in range(_NS - 1)}

    def step(ci, accs):
        if ci + _NS - 1 < _NCH:
            handles[ci + _NS - 1] = start_chunk(ci + _NS - 1)
        for h in handles.pop(ci):
            h.wait()
        return compute_chunk(ci, accs)

    def compute_chunk(ci, accs):
        pb, tb, _ = slots[ci % _NS]
        snapb = _OFFS[ci] // _SB

        def vec_body(vi, accs):
            accs = list(accs)
            s0 = vi * _SB
            loads, sqs = {}, {}

            def do_load(k):
                s = s0 + k * _L
                loads[k] = (pb[pl.ds(s, _L)], tb[pl.ds(s, _L)])

            def do_comp(k):
                p, t = loads.pop(k)
                d = p - t
                sqs[k] = d * d

            do_load(0)
            do_load(1)
            do_comp(0)
            for k in range(_U):
                if k + 2 < _U:
                    do_load(k + 2)
                if k + 1 < _U:
                    do_comp(k + 1)
                accs[k % 5] = accs[k % 5] + sqs.pop(k)
            s5 = ((accs[0] + accs[1]) + (accs[2] + accs[3])) + accs[4]
            snap[pl.ds((snapb + vi + 1) * _L, _L)] = s5
            return tuple(accs)

        return lax.fori_loop(0, _CHS[ci] // _SB, vec_body, accs)

    accs = tuple(zeros for _ in range(5))
    accs = step(0, accs)

    for h in probe_hs:
        h.wait()
    lvs = [plsc.load_gather(sampbuf, [(lane + _L * j) * _L])
           for j in range(_NPR // _L)]
    wo = []
    for g in range(1, _G):
        cnt = jnp.zeros((_L,), jnp.int32)
        for lv in lvs:
            cnt = cnt + jnp.where(lv < g, 1, 0)
        kg = jnp.clip(jnp.sum(cnt) - 1, 0, _NPR - 1)
        wo.append(jnp.minimum(kg * _STR, _PER_W - _WIN))
    win_hs = [pltpu.async_copy(lab_hbm.at[pl.ds(base + wo[i], _WIN)],
                               winbuf.at[pl.ds(i * _WIN, _WIN)], semx)
              for i in range(7)]

    accs = step(1, accs)

    for h in win_hs:
        h.wait()
    lane_off = jnp.where(lane < 7, lane * _WIN, 0)
    tgt = lane + 1
    lo = jnp.zeros((_L,), jnp.int32)
    hi = jnp.full((_L,), _WIN, jnp.int32)
    for _ in range(12):
        mid = (lo + hi) >> 1
        vals = plsc.load_gather(winbuf,
                                [lane_off + jnp.minimum(mid, _WIN - 1)])
        ge = (vals >= tgt) | (mid >= hi)
        hi = jnp.where(ge, mid, hi)
        lo = jnp.where(ge, lo, mid + 1)
    wo_vec = jnp.zeros((_L,), jnp.int32)
    for g in range(1, _G):
        wo_vec = jnp.where(lane == g - 1, wo[g - 1], wo_vec)
    b_vec = jnp.where(lane < 7, lo + wo_vec, 0)
    bscr[...] = b_vec
    bs, roffs = [], []
    ref_hs = []
    for g in range(1, _G):
        b = b_vec[g - 1]
        sb = b // _SB
        roff = jnp.minimum(sb, _NSNAP - 1) * _SB
        bs.append(b)
        roffs.append((sb, roff))
        ref_hs.append(pltpu.async_copy(
            p_hbm.at[pl.ds(base + roff, _SB)],
            rpbuf.at[pl.ds((g - 1) * _SB, _SB)], semx))
        ref_hs.append(pltpu.async_copy(
            t_hbm.at[pl.ds(base + roff, _SB)],
            rtbuf.at[pl.ds((g - 1) * _SB, _SB)], semx))

    for ci in range(2, _NCH):
        accs = step(ci, accs)

    for h in ref_hs:
        h.wait()
    for g in range(1, _G):
        b = bs[g - 1]
        sb, roff = roffs[g - 1]
        sb400 = sb * _SB
        acc_r = zeros
        for v in range(_U):
            rp = rpbuf[pl.ds((g - 1) * _SB + v * _L, _L)]
            rt = rtbuf[pl.ds((g - 1) * _SB + v * _L, _L)]
            d = rp - rt
            i = roff + v * _L + lane
            m = (i >= sb400) & (i < b)
            acc_r = acc_r + jnp.where(m, d * d, 0.0)
        snapvec = snap[pl.ds(sb * _L, _L)]
        sacc[pl.ds((g - 1) * _L, _L)] = snapvec + acc_r
    total = ((accs[0] + accs[1]) + (accs[2] + accs[3])) + accs[4]
    sacc[pl.ds(7 * _L, _L)] = total

    pltpu.sync_copy(sacc, sums_out.at[wid])
    pltpu.sync_copy(bscr, bnds_out.at[wid])


def _finalize_body(s_ref, b_ref, o_ref):
    s = jnp.sum(s_ref[...], axis=0, keepdims=True)
    bi = jnp.sum(b_ref[...], axis=0, keepdims=True)
    rowid = lax.broadcasted_iota(jnp.int32, (1, 128), 1) // _L
    laneid = lax.broadcasted_iota(jnp.int32, (1, _L), 1)
    P = [jnp.float32(0.0)]
    for g in range(1, _G):
        P.append(jnp.sum(jnp.where(rowid == g - 1, s, 0.0)))
    total = jnp.sum(jnp.where(rowid == 7, s, 0.0))
    P.append(total)
    B = [jnp.float32(0.0)]
    for g in range(1, _G):
        B.append(jnp.sum(jnp.where(laneid == g - 1, bi, 0))
                 .astype(jnp.float32))
    B.append(jnp.float32(_N))
    base = total / _N
    mse_sum = jnp.float32(0.0)
    mses = []
    for g in range(_G):
        m = (P[g + 1] - P[g]) / jnp.maximum(B[g + 1] - B[g], 1.0)
        mses.append(m)
        mse_sum = mse_sum + m
    mu = mse_sum / _G
    var = jnp.float32(0.0)
    for g in range(_G):
        dm = mses[g] - mu
        var = var + dm * dm
    var = var / (_G - 1)
    o_ref[...] = jnp.full((1, 1), base + var, jnp.float32)


_finalize = pl.pallas_call(
    _finalize_body,
    out_shape=jax.ShapeDtypeStruct((1, 1), jnp.float32),
)


def kernel(predictions, targets, group_labels):
    labels = group_labels.astype(jnp.int32)
    sums, bnds = _sc_partials(predictions, targets, labels)
    return _finalize(sums, bnds)[0, 0]


# speedup vs baseline: 1.0626x; 1.0626x over previous
import functools

import jax
import jax.numpy as jnp
from jax import lax
from jax.experimental import pallas as pl
from jax.experimental.pallas import tpu as pltpu
from jax.experimental.pallas import tpu_sc as plsc

_N = 3200000
_G = 8
_NW = 32
_PER_W = _N // _NW
_CH = 20000
_CHS = (4000, 16000, 20000, 20000, 20000, 20000)
_OFFS = tuple(sum(_CHS[:i]) for i in range(len(_CHS)))
_NCH = len(_CHS)
_L = 16
_U = 25
_STR = 1568
_NPR = 64
_WIN = 1600
_SB = _L * _U
_NSNAP = _PER_W // _SB

_mesh = plsc.VectorSubcoreMesh(core_axis_name="c", subcore_axis_name="s")


@functools.partial(
    pl.kernel,
    mesh=_mesh,
    compiler_params=pltpu.CompilerParams(needs_layout_passes=False),
    out_type=[
        jax.ShapeDtypeStruct((_NW, 128), jnp.float32),
        jax.ShapeDtypeStruct((_NW, _L), jnp.int32),
    ],
    scratch_types=[
        pltpu.VMEM((_CH,), jnp.float32),
        pltpu.VMEM((_CH,), jnp.float32),
        pltpu.VMEM((_CH,), jnp.float32),
        pltpu.VMEM((_CH,), jnp.float32),
        pltpu.VMEM((_NPR * _L,), jnp.int32),
        pltpu.VMEM((7 * _WIN,), jnp.int32),
        pltpu.VMEM(((_NSNAP + 6) * _L,), jnp.float32),
        pltpu.VMEM((7 * _SB,), jnp.float32),
        pltpu.VMEM((7 * _SB,), jnp.float32),
        pltpu.VMEM((_L,), jnp.int32),
        pltpu.VMEM((128,), jnp.float32),
        pltpu.SemaphoreType.DMA,
        pltpu.SemaphoreType.DMA,
        pltpu.SemaphoreType.DMA,
    ],
)
def _sc_partials(p_hbm, t_hbm, lab_hbm, sums_out, bnds_out,
                 pbuf0, tbuf0, pbuf1, tbuf1,
                 sampbuf, winbuf, snap, rpbuf, rtbuf, bscr, sacc,
                 sem0, sem1, semx):
    wid = lax.axis_index("s") * 2 + lax.axis_index("c")
    base = wid * _PER_W
    zeros = jnp.zeros((_L,), jnp.float32)
    lane = lax.iota(jnp.int32, _L)
    snap[pl.ds(0, _L)] = zeros

    probe_hs = [pltpu.async_copy(lab_hbm.at[pl.ds(base + k * _STR, _L)],
                                 sampbuf.at[pl.ds(k * _L, _L)], semx)
                for k in range(_NPR)]

    slots = ((pbuf0, tbuf0, sem0), (pbuf1, tbuf1, sem1))
    _NS = len(slots)

    def start_chunk(ci):
        pb, tb, sem = slots[ci % _NS]
        off = base + _OFFS[ci]
        sz = _CHS[ci]
        return (pltpu.async_copy(p_hbm.at[pl.ds(off, sz)],
                                 pb.at[pl.ds(0, sz)], sem),
                pltpu.async_copy(t_hbm.at[pl.ds(off, sz)],
                                 tb.at[pl.ds(0, sz)], sem))

    handles = {ci: start_chunk(ci) for ci in range(_NS - 1)}

    def step(ci, accs):
        if ci + _NS - 1 < _NCH:
            handles[ci + _NS - 1] = start_chunk(ci + _NS - 1)
        for h in handles.pop(ci):
            h.wait()
        return compute_chunk(ci, accs)

    def compute_chunk(ci, accs):
        pb, tb, _ = slots[ci % _NS]
        snapb = _OFFS[ci] // _SB

        def vec_body(vi, accs):
            accs = list(accs)
            s0 = vi * _SB
            loads, sqs = {}, {}

            def do_load(k):
                s = s0 + k * _L
                loads[k] = (pb[pl.ds(s, _L)], tb[pl.ds(s, _L)])

            def do_comp(k):
                p, t = loads.pop(k)
                d = p - t
                sqs[k] = d * d

            do_load(0)
            do_load(1)
            do_comp(0)
            for k in range(_U):
                if k + 2 < _U:
                    do_load(k + 2)
                if k + 1 < _U:
                    do_comp(k + 1)
                accs[k % 5] = accs[k % 5] + sqs.pop(k)
            s5 = ((accs[0] + accs[1]) + (accs[2] + accs[3])) + accs[4]
            snap[pl.ds((snapb + vi + 1) * _L, _L)] = s5
            return tuple(accs)

        return lax.fori_loop(0, _CHS[ci] // _SB, vec_body, accs)

    accs = tuple(zeros for _ in range(5))
    accs = step(0, accs)

    for h in probe_hs:
        h.wait()
    lvs = [plsc.load_gather(sampbuf, [(lane + _L * j) * _L])
           for j in range(_NPR // _L)]
    wo = []
    for g in range(1, _G):
        cnt = jnp.zeros((_L,), jnp.int32)
        for lv in lvs:
            cnt = cnt + jnp.where(lv < g, 1, 0)
        kg = jnp.clip(jnp.sum(cnt) - 1, 0, _NPR - 1)
        wo.append(jnp.minimum(kg * _STR, _PER_W - _WIN))
    win_hs = [pltpu.async_copy(lab_hbm.at[pl.ds(base + wo[i], _WIN)],
                               winbuf.at[pl.ds(i * _WIN, _WIN)], semx)
              for i in range(7)]

    accs = step(1, accs)

    for h in win_hs:
        h.wait()
    lane_off = jnp.where(lane < 7, lane * _WIN, 0)
    tgt = lane + 1
    lo = jnp.zeros((_L,), jnp.int32)
    hi = jnp.full((_L,), _WIN, jnp.int32)
    for _ in range(12):
        mid = (lo + hi) >> 1
        vals = plsc.load_gather(winbuf,
                                [lane_off + jnp.minimum(mid, _WIN - 1)])
        ge = (vals >= tgt) | (mid >= hi)
        hi = jnp.where(ge, mid, hi)
        lo = jnp.where(ge, lo, mid + 1)
    wo_vec = jnp.zeros((_L,), jnp.int32)
    for g in range(1, _G):
        wo_vec = jnp.where(lane == g - 1, wo[g - 1], wo_vec)
    b_vec = jnp.where(lane < 7, lo + wo_vec, 0)
    bscr[...] = b_vec
    bs, roffs = [], []
    ref_hs = []
    for g in range(1, _G):
        b = b_vec[g - 1]
        sb = b // _SB
        roff = jnp.minimum(sb, _NSNAP - 1) * _SB
        bs.append(b)
        roffs.append((sb, roff))
        ref_hs.append(pltpu.async_copy(
            p_hbm.at[pl.ds(base + roff, _SB)],
            rpbuf.at[pl.ds((g - 1) * _SB, _SB)], semx))
        ref_hs.append(pltpu.async_copy(
            t_hbm.at[pl.ds(base + roff, _SB)],
            rtbuf.at[pl.ds((g - 1) * _SB, _SB)], semx))

    for ci in range(2, _NCH):
        accs = step(ci, accs)

    for h in ref_hs:
        h.wait()
    for g in range(1, _G):
        b = bs[g - 1]
        sb, roff = roffs[g - 1]
        sb400 = sb * _SB
        acc_r = zeros
        for v in range(_U):
            rp = rpbuf[pl.ds((g - 1) * _SB + v * _L, _L)]
            rt = rtbuf[pl.ds((g - 1) * _SB + v * _L, _L)]
            d = rp - rt
            i = roff + v * _L + lane
            m = (i >= sb400) & (i < b)
            acc_r = acc_r + jnp.where(m, d * d, 0.0)
        snapvec = snap[pl.ds(sb * _L, _L)]
        sacc[pl.ds((g - 1) * _L, _L)] = snapvec + acc_r
    total = ((accs[0] + accs[1]) + (accs[2] + accs[3])) + accs[4]
    sacc[pl.ds(7 * _L, _L)] = total

    pltpu.sync_copy(sacc, sums_out.at[wid])
    pltpu.sync_copy(bscr, bnds_out.at[wid])


def _finalize_body(s_ref, b_ref, o_ref):
    s = jnp.sum(s_ref[...], axis=0, keepdims=True)
    bi = jnp.sum(b_ref[...], axis=0, keepdims=True)
    rowid = lax.broadcasted_iota(jnp.int32, (1, 128), 1) // _L
    laneid = lax.broadcasted_iota(jnp.int32, (1, _L), 1)
    P = [jnp.float32(0.0)]
    for g in range(1, _G):
        P.append(jnp.sum(jnp.where(rowid == g - 1, s, 0.0)))
    total = jnp.sum(jnp.where(rowid == 7, s, 0.0))
    P.append(total)
    B = [jnp.float32(0.0)]
    for g in range(1, _G):
        B.append(jnp.sum(jnp.where(laneid == g - 1, bi, 0))
                 .astype(jnp.float32))
    B.append(jnp.float32(_N))
    base = total / _N
    mse_sum = jnp.float32(0.0)
    mses = []
    for g in range(_G):
        m = (P[g + 1] - P[g]) / jnp.maximum(B[g + 1] - B[g], 1.0)
        mses.append(m)
        mse_sum = mse_sum + m
    mu = mse_sum / _G
    var = jnp.float32(0.0)
    for g in range(_G):
        dm = mses[g] - mu
        var = var + dm * dm
    var = var / (_G - 1)
    o_ref[...] = jnp.full((1, 1), base + var, jnp.float32)


_finalize = pl.pallas_call(
    _finalize_body,
    out_shape=jax.ShapeDtypeStruct((1, 1), jnp.float32),
)


def kernel(predictions, targets, group_labels):
    labels = group_labels.astype(jnp.int32)
    sums, bnds = _sc_partials(predictions, targets, labels)
    return _finalize(sums, bnds)[0, 0]


# speedup vs baseline: 1.0671x; 1.0042x over previous
import functools

import jax
import jax.numpy as jnp
from jax import lax
from jax.experimental import pallas as pl
from jax.experimental.pallas import tpu as pltpu
from jax.experimental.pallas import tpu_sc as plsc

_N = 3200000
_G = 8
_NW = 32
_PER_W = _N // _NW
_CH = 20000
_CHS = (4000, 16000, 20000, 20000, 20000, 20000)
_OFFS = tuple(sum(_CHS[:i]) for i in range(len(_CHS)))
_NCH = len(_CHS)
_L = 16
_U = 25
_STR = 1568
_NPR = 64
_WIN = 1600
_SB = _L * _U
_NSNAP = _PER_W // _SB

_mesh = plsc.VectorSubcoreMesh(core_axis_name="c", subcore_axis_name="s")


@functools.partial(
    pl.kernel,
    mesh=_mesh,
    compiler_params=pltpu.CompilerParams(needs_layout_passes=False),
    out_type=[
        jax.ShapeDtypeStruct((_NW, 128), jnp.float32),
        jax.ShapeDtypeStruct((_NW, _L), jnp.int32),
    ],
    scratch_types=[
        pltpu.VMEM((_CH,), jnp.float32),
        pltpu.VMEM((_CH,), jnp.float32),
        pltpu.VMEM((_CH,), jnp.float32),
        pltpu.VMEM((_CH,), jnp.float32),
        pltpu.VMEM((_NPR * _L,), jnp.int32),
        pltpu.VMEM((7 * _WIN,), jnp.int32),
        pltpu.VMEM(((_NSNAP + 6) * _L,), jnp.float32),
        pltpu.VMEM((7 * _SB,), jnp.float32),
        pltpu.VMEM((7 * _SB,), jnp.float32),
        pltpu.VMEM((_L,), jnp.int32),
        pltpu.VMEM((128,), jnp.float32),
        pltpu.SemaphoreType.DMA,
        pltpu.SemaphoreType.DMA,
        pltpu.SemaphoreType.DMA,
    ],
)
def _sc_partials(p_hbm, t_hbm, lab_hbm, sums_out, bnds_out,
                 pbuf0, tbuf0, pbuf1, tbuf1,
                 sampbuf, winbuf, snap, rpbuf, rtbuf, bscr, sacc,
                 sem0, sem1, semx):
    wid = lax.axis_index("s") * 2 + lax.axis_index("c")
    base = wid * _PER_W
    zeros = jnp.zeros((_L,), jnp.float32)
    lane = lax.iota(jnp.int32, _L)
    snap[pl.ds(0, _L)] = zeros

    probe_hs = [pltpu.async_copy(lab_hbm.at[pl.ds(base + k * _STR, _L)],
                                 sampbuf.at[pl.ds(k * _L, _L)], semx)
                for k in range(_NPR)]

    slots = ((pbuf0, tbuf0, sem0), (pbuf1, tbuf1, sem1))
    _NS = len(slots)

    def start_chunk(ci):
        pb, tb, sem = slots[ci % _NS]
        off = base + _OFFS[ci]
        sz = _CHS[ci]
        return (pltpu.async_copy(p_hbm.at[pl.ds(off, sz)],
                                 pb.at[pl.ds(0, sz)], sem),
                pltpu.async_copy(t_hbm.at[pl.ds(off, sz)],
                                 tb.at[pl.ds(0, sz)], sem))

    handles = {ci: start_chunk(ci) for ci in range(_NS - 1)}

    def step(ci, accs):
        if ci + _NS - 1 < _NCH:
            handles[ci + _NS - 1] = start_chunk(ci + _NS - 1)
        for h in handles.pop(ci):
            h.wait()
        return compute_chunk(ci, accs)

    def compute_chunk(ci, accs):
        pb, tb, _ = slots[ci % _NS]
        snapb = _OFFS[ci] // _SB

        def vec_body(vi, accs):
            accs = list(accs)
            s0 = vi * _SB
            loads, sqs = {}, {}

            def do_load(k):
                s = s0 + k * _L
                loads[k] = (pb[pl.ds(s, _L)], tb[pl.ds(s, _L)])

            def do_comp(k):
                p, t = loads.pop(k)
                d = p - t
                sqs[k] = d * d

            do_load(0)
            do_load(1)
            do_comp(0)
            for k in range(_U):
                if k + 2 < _U:
                    do_load(k + 2)
                if k + 1 < _U:
                    do_comp(k + 1)
                accs[k % 5] = accs[k % 5] + sqs.pop(k)
            s5 = ((accs[0] + accs[1]) + (accs[2] + accs[3])) + accs[4]
            snap[pl.ds((snapb + vi + 1) * _L, _L)] = s5
            return tuple(accs)

        return lax.fori_loop(0, _CHS[ci] // _SB, vec_body, accs)

    accs = tuple(zeros for _ in range(5))
    accs = step(0, accs)
    accs = step(1, accs)

    for h in probe_hs:
        h.wait()
    lvs = [plsc.load_gather(sampbuf, [(lane + _L * j) * _L])
           for j in range(_NPR // _L)]
    wo = []
    for g in range(1, _G):
        cnt = jnp.zeros((_L,), jnp.int32)
        for lv in lvs:
            cnt = cnt + jnp.where(lv < g, 1, 0)
        kg = jnp.clip(jnp.sum(cnt) - 1, 0, _NPR - 1)
        wo.append(jnp.minimum(kg * _STR, _PER_W - _WIN))
    win_hs = [pltpu.async_copy(lab_hbm.at[pl.ds(base + wo[i], _WIN)],
                               winbuf.at[pl.ds(i * _WIN, _WIN)], semx)
              for i in range(7)]

    accs = step(2, accs)

    for h in win_hs:
        h.wait()
    lane_off = jnp.where(lane < 7, lane * _WIN, 0)
    tgt = lane + 1
    lo = jnp.zeros((_L,), jnp.int32)
    hi = jnp.full((_L,), _WIN, jnp.int32)
    for _ in range(12):
        mid = (lo + hi) >> 1
        vals = plsc.load_gather(winbuf,
                                [lane_off + jnp.minimum(mid, _WIN - 1)])
        ge = (vals >= tgt) | (mid >= hi)
        hi = jnp.where(ge, mid, hi)
        lo = jnp.where(ge, lo, mid + 1)
    wo_vec = jnp.zeros((_L,), jnp.int32)
    for g in range(1, _G):
        wo_vec = jnp.where(lane == g - 1, wo[g - 1], wo_vec)
    b_vec = jnp.where(lane < 7, lo + wo_vec, 0)
    bscr[...] = b_vec
    bs, roffs = [], []
    ref_hs = []
    for g in range(1, _G):
        b = b_vec[g - 1]
        sb = b // _SB
        roff = jnp.minimum(sb, _NSNAP - 1) * _SB
        bs.append(b)
        roffs.append((sb, roff))
        ref_hs.append(pltpu.async_copy(
            p_hbm.at[pl.ds(base + roff, _SB)],
            rpbuf.at[pl.ds((g - 1) * _SB, _SB)], semx))
        ref_hs.append(pltpu.async_copy(
            t_hbm.at[pl.ds(base + roff, _SB)],
            rtbuf.at[pl.ds((g - 1) * _SB, _SB)], semx))

    for ci in range(3, _NCH):
        accs = step(ci, accs)

    for h in ref_hs:
        h.wait()
    for g in range(1, _G):
        b = bs[g - 1]
        sb, roff = roffs[g - 1]
        sb400 = sb * _SB
        acc_r = zeros
        for v in range(_U):
            rp = rpbuf[pl.ds((g - 1) * _SB + v * _L, _L)]
            rt = rtbuf[pl.ds((g - 1) * _SB + v * _L, _L)]
            d = rp - rt
            i = roff + v * _L + lane
            m = (i >= sb400) & (i < b)
            acc_r = acc_r + jnp.where(m, d * d, 0.0)
        snapvec = snap[pl.ds(sb * _L, _L)]
        sacc[pl.ds((g - 1) * _L, _L)] = snapvec + acc_r
    total = ((accs[0] + accs[1]) + (accs[2] + accs[3])) + accs[4]
    sacc[pl.ds(7 * _L, _L)] = total

    pltpu.sync_copy(sacc, sums_out.at[wid])
    pltpu.sync_copy(bscr, bnds_out.at[wid])


def _finalize_body(s_ref, b_ref, o_ref):
    s = jnp.sum(s_ref[...], axis=0, keepdims=True)
    bi = jnp.sum(b_ref[...], axis=0, keepdims=True)
    rowid = lax.broadcasted_iota(jnp.int32, (1, 128), 1) // _L
    laneid = lax.broadcasted_iota(jnp.int32, (1, _L), 1)
    P = [jnp.float32(0.0)]
    for g in range(1, _G):
        P.append(jnp.sum(jnp.where(rowid == g - 1, s, 0.0)))
    total = jnp.sum(jnp.where(rowid == 7, s, 0.0))
    P.append(total)
    B = [jnp.float32(0.0)]
    for g in range(1, _G):
        B.append(jnp.sum(jnp.where(laneid == g - 1, bi, 0))
                 .astype(jnp.float32))
    B.append(jnp.float32(_N))
    base = total / _N
    mse_sum = jnp.float32(0.0)
    mses = []
    for g in range(_G):
        m = (P[g + 1] - P[g]) / jnp.maximum(B[g + 1] - B[g], 1.0)
        mses.append(m)
        mse_sum = mse_sum + m
    mu = mse_sum / _G
    var = jnp.float32(0.0)
    for g in range(_G):
        dm = mses[g] - mu
        var = var + dm * dm
    var = var / (_G - 1)
    o_ref[...] = jnp.full((1, 1), base + var, jnp.float32)


_finalize = pl.pallas_call(
    _finalize_body,
    out_shape=jax.ShapeDtypeStruct((1, 1), jnp.float32),
)


def kernel(predictions, targets, group_labels):
    labels = group_labels.astype(jnp.int32)
    sums, bnds = _sc_partials(predictions, targets, labels)
    return _finalize(sums, bnds)[0, 0]
